# baseline (speedup 1.0000x reference)
import functools

import jax
import jax.numpy as jnp
from jax import lax
from jax.experimental import pallas as pl
from jax.experimental.pallas import tpu as pltpu
from jax.experimental.pallas import tpu_sc as plsc

_D = 256
_K = 8192
_B = 16
_N = 1024
_M = _B * _N

_BM = 256
_BK = 1024


def _dist_body(x2_ref, e2_ref, x_ref, e_ref, dist_hbm, ind_ref, nd_buf,
               out_sems):
    m = pl.program_id(0)
    xs = x_ref[...] * (-2.0)
    x2 = x2_ref[...]
    iota = lax.broadcasted_iota(jnp.int32, (_BM, _BK), 1).astype(jnp.float32)
    for kk in range(_K // _BK):
        sl = pl.ds(kk * _BK, _BK)
        xy2 = lax.dot_general(
            xs, e_ref[sl, :], (((1,), (1,)), ((), ())),
            preferred_element_type=jnp.float32)
        d2 = jnp.maximum((x2 + e2_ref[:, sl]) + xy2, 1.1754944e-38)
        nd = -(d2 * lax.rsqrt(d2))
        nd_buf[:, sl] = nd
        pltpu.make_async_copy(
            nd_buf.at[:, sl],
            dist_hbm.at[pl.ds(m * _BM, _BM), sl],
            out_sems.at[kk],
        ).start()
    maxv = None
    argm = None
    for kk in range(_K // _BK):
        nd = nd_buf[:, pl.ds(kk * _BK, _BK)]
        tmax = jnp.max(nd, axis=1, keepdims=True)
        tidx = jnp.min(jnp.where(nd == tmax, iota, jnp.float32(_K)),
                       axis=1, keepdims=True) + jnp.float32(kk * _BK)
        if kk == 0:
            maxv, argm = tmax, tidx
        else:
            upd = tmax > maxv
            maxv = jnp.where(upd, tmax, maxv)
            argm = jnp.where(upd, tidx, argm)
    ind_ref[...] = argm.astype(jnp.int32)
    for kk in range(_K // _BK):
        pltpu.make_async_copy(
            nd_buf.at[:, pl.ds(kk * _BK, _BK)],
            dist_hbm.at[pl.ds(m * _BM, _BM), pl.ds(kk * _BK, _BK)],
            out_sems.at[kk],
        ).wait()


_SC_NC = 2
_SC_NS = 16
_NW = _SC_NC * _SC_NS
_BPW = _M // _NW
_CH = 128


def _gather_quantize(idx, table):
    mesh = plsc.VectorSubcoreMesh(core_axis_name="c", subcore_axis_name="s")

    n_ch = _BPW // _CH

    @functools.partial(
        pl.kernel, mesh=mesh,
        out_type=jax.ShapeDtypeStruct((_M, _D), jnp.float32),
        scratch_types=[
            pltpu.VMEM((_BPW,), jnp.int32),
            pltpu.VMEM((_CH, _D), jnp.float32),
            pltpu.VMEM((_CH, _D), jnp.float32),
            pltpu.SemaphoreType.DMA,
            pltpu.SemaphoreType.DMA,
            pltpu.SemaphoreType.DMA,
            pltpu.SemaphoreType.DMA,
        ],
    )
    def gk(idx_hbm, table_hbm, out_hbm, idx_v, rows0, rows1, g0, g1, s0, s1):
        wid = lax.axis_index("s") * _SC_NC + lax.axis_index("c")
        base = wid * _BPW
        bufs = (rows0, rows1)
        gsem = (g0, g1)
        ssem = (s0, s1)
        pltpu.sync_copy(idx_hbm.at[pl.ds(base, _BPW)], idx_v)
        gath = [None, None]
        stor = [None, None]
        for c in range(n_ch):
            b = c % 2
            if stor[b] is not None:
                stor[b].wait()
            gath[b] = pltpu.async_copy(
                table_hbm.at[idx_v.at[pl.ds(c * _CH, _CH)]], bufs[b], gsem[b])
            if c > 0:
                pb = (c - 1) % 2
                gath[pb].wait()
                stor[pb] = pltpu.async_copy(
                    bufs[pb], out_hbm.at[pl.ds(base + (c - 1) * _CH, _CH)],
                    ssem[pb])
        lb = (n_ch - 1) % 2
        gath[lb].wait()
        stor[lb] = pltpu.async_copy(
            bufs[lb], out_hbm.at[pl.ds(base + (n_ch - 1) * _CH, _CH)],
            ssem[lb])
        stor[(n_ch - 2) % 2].wait()
        stor[lb].wait()

    return gk(idx, table)


def kernel(x, embed):
    xf = x.astype(jnp.float32).reshape(_M, _D)
    e0 = embed[0]
    x2 = jnp.sum(xf ** 2, axis=-1).reshape(_M, 1)
    e2 = jnp.sum(e0 ** 2, axis=-1).reshape(1, _K)

    dist2d, ind2d = pl.pallas_call(
        _dist_body,
        grid=(_M // _BM,),
        in_specs=[
            pl.BlockSpec((_BM, 1), lambda m: (m, 0)),
            pl.BlockSpec((1, _K), lambda m: (0, 0)),
            pl.BlockSpec((_BM, _D), lambda m: (m, 0)),
            pl.BlockSpec((_K, _D), lambda m: (0, 0)),
        ],
        out_specs=[
            pl.BlockSpec(memory_space=pl.ANY),
            pl.BlockSpec((_BM, 1), lambda m: (m, 0)),
        ],
        out_shape=[
            jax.ShapeDtypeStruct((_M, _K), jnp.float32),
            jax.ShapeDtypeStruct((_M, 1), jnp.int32),
        ],
        scratch_shapes=[
            pltpu.VMEM((_BM, _K), jnp.float32),
            pltpu.SemaphoreType.DMA((_K // _BK,)),
        ],
        compiler_params=pltpu.CompilerParams(
            dimension_semantics=("parallel",)),
    )(x2, e2, xf, e0)

    ind_flat = ind2d.reshape(_M)
    quantize = _gather_quantize(ind_flat, e0).reshape(_B, _N, _D)
    embed_ind = ind_flat.reshape(_B, _N)
    dist = dist2d.reshape(1, _B, _N, _K)
    return quantize, embed_ind, dist


# speedup vs baseline: 1.1256x; 1.1256x over previous
import functools

import jax
import jax.numpy as jnp
from jax import lax
from jax.experimental import pallas as pl
from jax.experimental.pallas import tpu as pltpu
from jax.experimental.pallas import tpu_sc as plsc

_D = 256
_K = 8192
_B = 16
_N = 1024
_M = _B * _N

_BM = 256
_BK = 2048


def _dist_body(x2_ref, e2_ref, x_ref, e_ref, dist_ref, ind_ref):
    xs = x_ref[...] * (-2.0)
    x2 = x2_ref[...]
    iota = lax.broadcasted_iota(jnp.int32, (_BM, _BK), 1).astype(jnp.float32)
    for kk in range(_K // _BK):
        sl = pl.ds(kk * _BK, _BK)
        xy2 = lax.dot_general(
            xs, e_ref[sl, :], (((1,), (1,)), ((), ())),
            preferred_element_type=jnp.float32)
        d2 = jnp.maximum((x2 + e2_ref[:, sl]) + xy2, 1.1754944e-38)
        nd = -(d2 * lax.rsqrt(d2))
        dist_ref[:, sl] = nd
    maxv = None
    argm = None
    for kk in range(_K // _BK):
        nd = dist_ref[:, pl.ds(kk * _BK, _BK)]
        tmax = jnp.max(nd, axis=1, keepdims=True)
        tidx = jnp.min(jnp.where(nd == tmax, iota, jnp.float32(_K)),
                       axis=1, keepdims=True) + jnp.float32(kk * _BK)
        if kk == 0:
            maxv, argm = tmax, tidx
        else:
            upd = tmax > maxv
            maxv = jnp.where(upd, tmax, maxv)
            argm = jnp.where(upd, tidx, argm)
    ind_ref[...] = argm.astype(jnp.int32)


_SC_NC = 2
_SC_NS = 16
_NW = _SC_NC * _SC_NS
_BPW = _M // _NW
_CH = 128


def _gather_quantize(idx, table):
    mesh = plsc.VectorSubcoreMesh(core_axis_name="c", subcore_axis_name="s")

    n_ch = _BPW // _CH

    @functools.partial(
        pl.kernel, mesh=mesh,
        out_type=jax.ShapeDtypeStruct((_M, _D), jnp.float32),
        scratch_types=[
            pltpu.VMEM((_BPW,), jnp.int32),
            pltpu.VMEM((_CH, _D), jnp.float32),
            pltpu.VMEM((_CH, _D), jnp.float32),
            pltpu.SemaphoreType.DMA,
            pltpu.SemaphoreType.DMA,
            pltpu.SemaphoreType.DMA,
            pltpu.SemaphoreType.DMA,
        ],
    )
    def gk(idx_hbm, table_hbm, out_hbm, idx_v, rows0, rows1, g0, g1, s0, s1):
        wid = lax.axis_index("s") * _SC_NC + lax.axis_index("c")
        base = wid * _BPW
        bufs = (rows0, rows1)
        gsem = (g0, g1)
        ssem = (s0, s1)
        pltpu.sync_copy(idx_hbm.at[pl.ds(base, _BPW)], idx_v)
        gath = [None, None]
        stor = [None, None]
        for c in range(n_ch):
            b = c % 2
            if stor[b] is not None:
                stor[b].wait()
            gath[b] = pltpu.async_copy(
                table_hbm.at[idx_v.at[pl.ds(c * _CH, _CH)]], bufs[b], gsem[b])
            if c > 0:
                pb = (c - 1) % 2
                gath[pb].wait()
                stor[pb] = pltpu.async_copy(
                    bufs[pb], out_hbm.at[pl.ds(base + (c - 1) * _CH, _CH)],
                    ssem[pb])
        lb = (n_ch - 1) % 2
        gath[lb].wait()
        stor[lb] = pltpu.async_copy(
            bufs[lb], out_hbm.at[pl.ds(base + (n_ch - 1) * _CH, _CH)],
            ssem[lb])
        stor[(n_ch - 2) % 2].wait()
        stor[lb].wait()

    return gk(idx, table)


def kernel(x, embed):
    xf = x.astype(jnp.float32).reshape(_M, _D)
    e0 = embed[0]
    x2 = jnp.sum(xf ** 2, axis=-1).reshape(_M, 1)
    e2 = jnp.sum(e0 ** 2, axis=-1).reshape(1, _K)

    dist2d, ind2d = pl.pallas_call(
        _dist_body,
        grid=(_M // _BM,),
        in_specs=[
            pl.BlockSpec((_BM, 1), lambda m: (m, 0)),
            pl.BlockSpec((1, _K), lambda m: (0, 0)),
            pl.BlockSpec((_BM, _D), lambda m: (m, 0)),
            pl.BlockSpec((_K, _D), lambda m: (0, 0)),
        ],
        out_specs=[
            pl.BlockSpec((_BM, _K), lambda m: (m, 0)),
            pl.BlockSpec((_BM, 1), lambda m: (m, 0)),
        ],
        out_shape=[
            jax.ShapeDtypeStruct((_M, _K), jnp.float32),
            jax.ShapeDtypeStruct((_M, 1), jnp.int32),
        ],
        compiler_params=pltpu.CompilerParams(
            dimension_semantics=("parallel",)),
    )(x2, e2, xf, e0)

    ind_flat = ind2d.reshape(_M)
    quantize = _gather_quantize(ind_flat, e0).reshape(_B, _N, _D)
    embed_ind = ind_flat.reshape(_B, _N)
    dist = dist2d.reshape(1, _B, _N, _K)
    return quantize, embed_ind, dist
